# baseline (speedup 1.0000x reference)
import jax
import jax.numpy as jnp
from jax.experimental import pallas as pl
from jax.experimental.pallas import tpu as pltpu

_B, _T, _N = 256, 16, 1024
_NSTAT = 10
_T_ITERS = 20
_FILTER_POS = 512.0


def _sigm(v):
    return 0.5 * jnp.tanh(0.5 * v) + 0.5


def _gcn_body(params_ref, adjT_ref, h0_ref, h1_ref, xs_hbm, m_hbm, out_ref,
              xs_v, m10_v, ml_v, sem_xs, sem_m10, sem_ml):
    cp_xs = pltpu.make_async_copy(xs_hbm, xs_v, sem_xs)
    cp_m10 = pltpu.make_async_copy(m_hbm.at[:, : _NSTAT * _N], m10_v, sem_m10)
    cp_ml = pltpu.make_async_copy(
        m_hbm.at[:, (_T - 1) * _N:], ml_v, sem_ml)
    cp_xs.start()
    cp_m10.start()
    cp_ml.start()

    c1 = params_ref[0]
    c2 = params_ref[1]
    ca = params_ref[2]
    wf = params_ref[3]
    bf = params_ref[4]

    adjT = adjT_ref[...]
    rs = jnp.sum(adjT.astype(jnp.float32), axis=0, keepdims=True)
    crs = c2 * rs

    def dot(a, b):
        return jax.lax.dot_general(
            a, b, (((1,), (0,)), ((), ())), preferred_element_type=jnp.float32)

    h0 = h0_ref[...]
    h1 = h1_ref[...]

    t0 = dot(h0.astype(jnp.bfloat16), adjT)
    t1 = dot(h1.astype(jnp.bfloat16), adjT)
    d = jnp.log(h0 + 1e-10) - jnp.log(h1 + 1e-10) - (c1 * t0 + c2 * t1)

    for _ in range(_T_ITERS - 1):
        t0 = dot(_sigm(d).astype(jnp.bfloat16), adjT)
        d = d - ca * t0 - crs
    h1 = _sigm(-d)

    lane = jax.lax.broadcasted_iota(jnp.int32, (1, _N), 1).astype(jnp.float32)
    w = _sigm(lane - _FILTER_POS)

    cp_ml.wait()
    mwl = ml_v[...] * w
    mean_cur = jnp.sum(h1 * mwl, axis=1) / (jnp.sum(mwl, axis=1) + 1e-10)

    cp_xs.wait()
    cp_m10.wait()
    stats = []
    for t in range(_NSTAT):
        sl = pl.ds(t * _N, _N)
        xs_t = xs_v[:, sl].astype(jnp.float32)
        mw_t = m10_v[:, sl] * w
        num = jnp.sum(xs_t * mw_t, axis=1)
        den = jnp.sum(mw_t, axis=1) + 1e-10
        stats.append(num / den)
    mean = sum(stats) / _NSTAT
    sumsq = sum((s - mean) ** 2 for s in stats)
    std = jnp.sqrt(sumsq / (_NSTAT - 1))

    z = (mean_cur - mean) / (std + 1e-6)
    out_ref[...] = _sigm(z * wf + bf)


def kernel(x, adj_in, m, w1, w11, w_prime1, w_prime11, w2, w22, w_prime2,
           w_prime22, w_final, b_final):
    a = jnp.abs(w1[0])
    b = jnp.abs(w_prime1[0])
    c = jnp.abs(w_prime11[0])
    dd = jnp.abs(w11[0])
    c1 = -(a + b)
    c2 = c + dd
    ca = c1 - c2
    params = jnp.stack([c1, c2, ca, w_final[0], b_final[0],
                        jnp.float32(0), jnp.float32(0), jnp.float32(0)])

    adjT = adj_in.T.astype(jnp.bfloat16)
    h0 = x[:, -1, :, 0]
    h1 = x[:, -1, :, 1]
    xs = x[:, :_NSTAT, :, 1].astype(jnp.bfloat16).reshape(_B, _NSTAT * _N)
    m2 = m.reshape(_B, _T * _N)

    out = pl.pallas_call(
        _gcn_body,
        out_shape=jax.ShapeDtypeStruct((_B,), jnp.float32),
        in_specs=[
            pl.BlockSpec(memory_space=pltpu.SMEM),
            pl.BlockSpec(memory_space=pltpu.VMEM),
            pl.BlockSpec(memory_space=pltpu.VMEM),
            pl.BlockSpec(memory_space=pltpu.VMEM),
            pl.BlockSpec(memory_space=pltpu.MemorySpace.HBM),
            pl.BlockSpec(memory_space=pltpu.MemorySpace.HBM),
        ],
        out_specs=pl.BlockSpec(memory_space=pltpu.VMEM),
        scratch_shapes=[
            pltpu.VMEM((_B, _NSTAT * _N), jnp.bfloat16),
            pltpu.VMEM((_B, _NSTAT * _N), jnp.float32),
            pltpu.VMEM((_B, _N), jnp.float32),
            pltpu.SemaphoreType.DMA,
            pltpu.SemaphoreType.DMA,
            pltpu.SemaphoreType.DMA,
        ],
        compiler_params=pltpu.CompilerParams(
            vmem_limit_bytes=100 * 1024 * 1024),
    )(params, adjT, h0, h1, xs, m2)
    return out


# speedup vs baseline: 1.1612x; 1.1612x over previous
import jax
import jax.numpy as jnp
from jax.experimental import pallas as pl
from jax.experimental.pallas import tpu as pltpu

_B, _T, _N = 256, 16, 1024
_NSTAT = 10
_T_ITERS = 20
_FILTER_POS = 512.0


def _sigm(v):
    return 0.5 * jnp.tanh(0.5 * v) + 0.5


def _gcn_body(params_ref, adj_ref, h0_ref, h1_ref, xs_hbm, m_hbm, out_ref,
              xs_v, m10_v, ml_v, sem_xs, sem_m10, sem_ml):
    cp_xs = pltpu.make_async_copy(xs_hbm, xs_v, sem_xs)
    cp_m10 = pltpu.make_async_copy(m_hbm.at[:, : _NSTAT * _N], m10_v, sem_m10)
    cp_ml = pltpu.make_async_copy(
        m_hbm.at[:, (_T - 1) * _N:], ml_v, sem_ml)
    cp_xs.start()
    cp_m10.start()
    cp_ml.start()

    c1 = params_ref[0]
    c2 = params_ref[1]
    ca = params_ref[2]
    wf = params_ref[3]
    bf = params_ref[4]

    adj = adj_ref[...]
    rs = jnp.sum(adj, axis=1, keepdims=True).T
    crs = c2 * rs
    adjT = adj.astype(jnp.bfloat16).T

    def dot(a, b):
        return jax.lax.dot_general(
            a, b, (((1,), (0,)), ((), ())), preferred_element_type=jnp.float32)

    h0 = h0_ref[...]
    h1 = h1_ref[...]

    t0 = dot(h0.astype(jnp.bfloat16), adjT)
    t1 = dot(h1.astype(jnp.bfloat16), adjT)
    d = jnp.log(h0 + 1e-10) - jnp.log(h1 + 1e-10) - (c1 * t0 + c2 * t1)

    for _ in range(_T_ITERS - 1):
        t0 = dot(_sigm(d).astype(jnp.bfloat16), adjT)
        d = d - ca * t0 - crs
    h1 = _sigm(-d)

    lane = jax.lax.broadcasted_iota(jnp.int32, (1, _N), 1).astype(jnp.float32)
    w = _sigm(lane - _FILTER_POS)

    cp_ml.wait()
    mwl = ml_v[...] * w
    mean_cur = jnp.sum(h1 * mwl, axis=1) / (jnp.sum(mwl, axis=1) + 1e-10)

    cp_xs.wait()
    cp_m10.wait()
    stats = []
    for t in range(_NSTAT):
        sl = pl.ds(t * _N, _N)
        xs_t = xs_v[:, sl]
        mw_t = m10_v[:, sl] * w
        num = jnp.sum(xs_t * mw_t, axis=1)
        den = jnp.sum(mw_t, axis=1) + 1e-10
        stats.append(num / den)
    mean = sum(stats) / _NSTAT
    sumsq = sum((s - mean) ** 2 for s in stats)
    std = jnp.sqrt(sumsq / (_NSTAT - 1))

    z = (mean_cur - mean) / (std + 1e-6)
    out_ref[...] = _sigm(z * wf + bf)


def kernel(x, adj_in, m, w1, w11, w_prime1, w_prime11, w2, w22, w_prime2,
           w_prime22, w_final, b_final):
    a = jnp.abs(w1[0])
    b = jnp.abs(w_prime1[0])
    c = jnp.abs(w_prime11[0])
    dd = jnp.abs(w11[0])
    c1 = -(a + b)
    c2 = c + dd
    ca = c1 - c2
    params = jnp.stack([c1, c2, ca, w_final[0], b_final[0],
                        jnp.float32(0), jnp.float32(0), jnp.float32(0)])

    h0 = x[:, -1, :, 0]
    h1 = x[:, -1, :, 1]
    xs = x[:, :_NSTAT, :, 1].reshape(_B, _NSTAT * _N)
    m2 = m.reshape(_B, _T * _N)

    out = pl.pallas_call(
        _gcn_body,
        out_shape=jax.ShapeDtypeStruct((_B,), jnp.float32),
        in_specs=[
            pl.BlockSpec(memory_space=pltpu.SMEM),
            pl.BlockSpec(memory_space=pltpu.VMEM),
            pl.BlockSpec(memory_space=pltpu.VMEM),
            pl.BlockSpec(memory_space=pltpu.VMEM),
            pl.BlockSpec(memory_space=pltpu.MemorySpace.HBM),
            pl.BlockSpec(memory_space=pltpu.MemorySpace.HBM),
        ],
        out_specs=pl.BlockSpec(memory_space=pltpu.VMEM),
        scratch_shapes=[
            pltpu.VMEM((_B, _NSTAT * _N), jnp.float32),
            pltpu.VMEM((_B, _NSTAT * _N), jnp.float32),
            pltpu.VMEM((_B, _N), jnp.float32),
            pltpu.SemaphoreType.DMA,
            pltpu.SemaphoreType.DMA,
            pltpu.SemaphoreType.DMA,
        ],
        compiler_params=pltpu.CompilerParams(
            vmem_limit_bytes=100 * 1024 * 1024),
    )(params, adj_in, h0, h1, xs, m2)
    return out


# speedup vs baseline: 1.4916x; 1.2846x over previous
import jax
import jax.numpy as jnp
from jax.experimental import pallas as pl
from jax.experimental.pallas import tpu as pltpu

_B, _T, _N = 256, 16, 1024
_NSTAT = 10
_T_ITERS = 20
_FILTER_POS = 512.0


def _gcn_body(params_ref, adj_ref, h0_ref, h1_ref, xs_hbm, m_hbm, out_ref,
              xs_v, m_v, sem_xs, sem_m):
    cp_xs = pltpu.make_async_copy(xs_hbm, xs_v, sem_xs)
    cp_m = pltpu.make_async_copy(m_hbm, m_v, sem_m)
    cp_xs.start()
    cp_m.start()

    c1 = params_ref[0]
    c2 = params_ref[1]
    ca = params_ref[2]
    wf = params_ref[3]
    bf = params_ref[4]

    adj = adj_ref[...]
    rs = jnp.sum(adj, axis=1, keepdims=True).T
    crs = c2 * rs
    adjT = adj.astype(jnp.bfloat16).T

    def dot(a, b):
        return jax.lax.dot_general(
            a, b, (((1,), (0,)), ((), ())), preferred_element_type=jnp.float32)

    h0 = h0_ref[...]
    h1 = h1_ref[...]

    t0 = dot(h0.astype(jnp.bfloat16), adjT)
    t1 = dot(h1.astype(jnp.bfloat16), adjT)
    d = jnp.log(h0 + 1e-10) - jnp.log(h1 + 1e-10) - (c1 * t0 + c2 * t1)

    for _ in range(_T_ITERS - 1):
        t0 = dot(jax.nn.sigmoid(d).astype(jnp.bfloat16), adjT)
        d = d - ca * t0 - crs
    h1 = jax.nn.sigmoid(-d)

    lane = jax.lax.broadcasted_iota(jnp.int32, (1, _N), 1).astype(jnp.float32)
    w = jax.nn.sigmoid(lane - _FILTER_POS)

    cp_m.wait()
    mwl = m_v[:, _T - 1, :] * w
    mean_cur = jnp.sum(h1 * mwl, axis=1) / (jnp.sum(mwl, axis=1) + 1e-10)

    cp_xs.wait()
    xs = xs_v[...]
    mw = m_v[:, : _NSTAT, :] * w[None]
    num = jnp.sum(xs * mw, axis=2)
    den = jnp.sum(mw, axis=2) + 1e-10
    stat10 = num / den
    mean = jnp.mean(stat10, axis=1)
    std = jnp.sqrt(jnp.sum((stat10 - mean[:, None]) ** 2, axis=1) / (_NSTAT - 1))

    z = (mean_cur - mean) / (std + 1e-6)
    out_ref[...] = jax.nn.sigmoid(z * wf + bf)


def kernel(x, adj_in, m, w1, w11, w_prime1, w_prime11, w2, w22, w_prime2,
           w_prime22, w_final, b_final):
    a = jnp.abs(w1[0])
    b = jnp.abs(w_prime1[0])
    c = jnp.abs(w_prime11[0])
    dd = jnp.abs(w11[0])
    c1 = -(a + b)
    c2 = c + dd
    ca = c1 - c2
    params = jnp.stack([c1, c2, ca, w_final[0], b_final[0],
                        jnp.float32(0), jnp.float32(0), jnp.float32(0)])

    h0 = x[:, -1, :, 0]
    h1 = x[:, -1, :, 1]
    xs = x[:, :_NSTAT, :, 1]

    out = pl.pallas_call(
        _gcn_body,
        out_shape=jax.ShapeDtypeStruct((_B,), jnp.float32),
        in_specs=[
            pl.BlockSpec(memory_space=pltpu.SMEM),
            pl.BlockSpec(memory_space=pltpu.VMEM),
            pl.BlockSpec(memory_space=pltpu.VMEM),
            pl.BlockSpec(memory_space=pltpu.VMEM),
            pl.BlockSpec(memory_space=pltpu.MemorySpace.HBM),
            pl.BlockSpec(memory_space=pltpu.MemorySpace.HBM),
        ],
        out_specs=pl.BlockSpec(memory_space=pltpu.VMEM),
        scratch_shapes=[
            pltpu.VMEM((_B, _NSTAT, _N), jnp.float32),
            pltpu.VMEM((_B, _T, _N), jnp.float32),
            pltpu.SemaphoreType.DMA,
            pltpu.SemaphoreType.DMA,
        ],
        compiler_params=pltpu.CompilerParams(
            vmem_limit_bytes=100 * 1024 * 1024),
    )(params, adj_in, h0, h1, xs, m)
    return out


# speedup vs baseline: 1.5118x; 1.0135x over previous
import jax
import jax.numpy as jnp
from jax.experimental import pallas as pl
from jax.experimental.pallas import tpu as pltpu

_B, _T, _N = 256, 16, 1024
_NSTAT = 10
_T_ITERS = 20
_FILTER_POS = 512.0


def _sigm(v):
    return 0.5 * jnp.tanh(0.5 * v) + 0.5


def _gcn_body(params_ref, adj_ref, h0_ref, h1_ref, xs_hbm, m_hbm, out_ref,
              xs_v, m_v, sem_xs, sem_m):
    cp_xs = pltpu.make_async_copy(xs_hbm, xs_v, sem_xs)
    cp_m = pltpu.make_async_copy(m_hbm, m_v, sem_m)
    cp_xs.start()
    cp_m.start()

    c1 = params_ref[0]
    c2 = params_ref[1]
    ca = params_ref[2]
    wf = params_ref[3]
    bf = params_ref[4]

    adj = adj_ref[...]
    rs = jnp.sum(adj, axis=1, keepdims=True).T
    crs = c2 * rs
    adjT = adj.astype(jnp.bfloat16).T

    def dot(a, b):
        return jax.lax.dot_general(
            a, b, (((1,), (0,)), ((), ())), preferred_element_type=jnp.float32)

    h0 = h0_ref[...]
    h1 = h1_ref[...]

    t0 = dot(h0.astype(jnp.bfloat16), adjT)
    t1 = dot(h1.astype(jnp.bfloat16), adjT)
    d = jnp.log(h0 + 1e-10) - jnp.log(h1 + 1e-10) - (c1 * t0 + c2 * t1)

    for _ in range(_T_ITERS - 1):
        t0 = dot(_sigm(d).astype(jnp.bfloat16), adjT)
        d = d - ca * t0 - crs
    h1 = _sigm(-d)

    lane = jax.lax.broadcasted_iota(jnp.int32, (1, _N), 1).astype(jnp.float32)
    w = jax.nn.sigmoid(lane - _FILTER_POS)

    cp_m.wait()
    mwl = m_v[:, _T - 1, :] * w
    mean_cur = jnp.sum(h1 * mwl, axis=1) / (jnp.sum(mwl, axis=1) + 1e-10)

    cp_xs.wait()
    xs = xs_v[...]
    mw = m_v[:, : _NSTAT, :] * w[None]
    num = jnp.sum(xs * mw, axis=2)
    den = jnp.sum(mw, axis=2) + 1e-10
    stat10 = num / den
    mean = jnp.mean(stat10, axis=1)
    std = jnp.sqrt(jnp.sum((stat10 - mean[:, None]) ** 2, axis=1) / (_NSTAT - 1))

    z = (mean_cur - mean) / (std + 1e-6)
    out_ref[...] = jax.nn.sigmoid(z * wf + bf)


def kernel(x, adj_in, m, w1, w11, w_prime1, w_prime11, w2, w22, w_prime2,
           w_prime22, w_final, b_final):
    a = jnp.abs(w1[0])
    b = jnp.abs(w_prime1[0])
    c = jnp.abs(w_prime11[0])
    dd = jnp.abs(w11[0])
    c1 = -(a + b)
    c2 = c + dd
    ca = c1 - c2
    params = jnp.stack([c1, c2, ca, w_final[0], b_final[0],
                        jnp.float32(0), jnp.float32(0), jnp.float32(0)])

    h0 = x[:, -1, :, 0]
    h1 = x[:, -1, :, 1]
    xs = x[:, :_NSTAT, :, 1]

    out = pl.pallas_call(
        _gcn_body,
        out_shape=jax.ShapeDtypeStruct((_B,), jnp.float32),
        in_specs=[
            pl.BlockSpec(memory_space=pltpu.SMEM),
            pl.BlockSpec(memory_space=pltpu.VMEM),
            pl.BlockSpec(memory_space=pltpu.VMEM),
            pl.BlockSpec(memory_space=pltpu.VMEM),
            pl.BlockSpec(memory_space=pltpu.MemorySpace.HBM),
            pl.BlockSpec(memory_space=pltpu.MemorySpace.HBM),
        ],
        out_specs=pl.BlockSpec(memory_space=pltpu.VMEM),
        scratch_shapes=[
            pltpu.VMEM((_B, _NSTAT, _N), jnp.float32),
            pltpu.VMEM((_B, _T, _N), jnp.float32),
            pltpu.SemaphoreType.DMA,
            pltpu.SemaphoreType.DMA,
        ],
        compiler_params=pltpu.CompilerParams(
            vmem_limit_bytes=100 * 1024 * 1024),
    )(params, adj_in, h0, h1, xs, m)
    return out
